# baseline (speedup 1.0000x reference)
import functools

import jax
import jax.numpy as jnp
import numpy as np
from jax import lax
from jax.experimental import pallas as pl
from jax.experimental.pallas import tpu as pltpu
from jax.experimental.pallas import tpu_sc as plsc

K_TOP = 30
CHUNK = 128
NCH = 64
INTMIN = np.int32(-(2**31))


def _encode_body(x_ref, w_ref, benc_ref, bdec_ref, pre_ref):
    xc = x_ref[...] - bdec_ref[...]
    acc = lax.dot_general(
        xc, w_ref[...], (((1,), (1,)), ((), ())),
        preferred_element_type=jnp.float32,
    )
    pre_ref[...] = acc + benc_ref[...]


def _chunkmax_body(pre_ref, m_ref):
    p = pre_ref[...]
    br, bf = p.shape
    m_ref[...] = jnp.max(p.reshape(br, bf // CHUNK, CHUNK), axis=2)


def _decode_body(pre_ref, t_ref, wdt_ref, bdec_ref, sparse_ref, recon_ref):
    j = pl.program_id(1)
    p = pre_ref[...]
    s = jnp.where(p >= t_ref[...], jnp.maximum(p, 0.0), 0.0)
    sparse_ref[...] = s
    contrib = lax.dot_general(
        s, wdt_ref[...], (((1,), (0,)), ((), ())),
        preferred_element_type=jnp.float32,
    )

    @pl.when(j == 0)
    def _():
        recon_ref[...] = contrib + bdec_ref[...]

    @pl.when(j != 0)
    def _():
        recon_ref[...] += contrib


def _f32_to_key(v):
    b = plsc.bitcast(v, jnp.int32)
    return jnp.where(b < 0, jnp.bitwise_xor(jnp.bitwise_not(b), INTMIN), b)


def _key_to_f32(s):
    b = jnp.where(s < 0, jnp.bitwise_not(jnp.bitwise_xor(s, INTMIN)), s)
    return plsc.bitcast(b, jnp.float32)


def _zero_hist(hist):
    z = jnp.zeros((16,), jnp.int32)
    for b in range(16):
        hist[pl.ds(b * 16, 16)] = z


def _pick_bucket(hist, lane, kk):
    def body(b2, carry):
        cum, bucket, kk2, found = carry
        b = 15 - b2
        hb = plsc.load_gather(hist, [lane * 16 + b])
        cum2 = cum + hb
        hit = jnp.logical_and(jnp.logical_not(found), cum2 >= kk)
        bucket = jnp.where(hit, b, bucket)
        kk2 = jnp.where(hit, kk - cum, kk2)
        return cum2, bucket, kk2, jnp.logical_or(found, hit)

    z = jnp.zeros((16,), jnp.int32)
    _, bucket, kk2, _ = lax.fori_loop(
        0, 16, body, (z, z, z, jnp.zeros((16,), jnp.bool_)))
    return bucket, kk2


def _radix_select(load0, n0, kk0, sa, sb, hist, lane):
    ones = jnp.ones((16,), jnp.int32)
    kk = jnp.full((16,), kk0, jnp.int32)

    _zero_hist(hist)

    def h0(i, c):
        for u in range(4):
            s = load0(i * 4 + u)
            d = jnp.bitwise_xor((s >> 28) & 15, 8)
            plsc.addupdate_scatter(hist, [lane * 16 + d], ones)
        return c

    lax.fori_loop(0, n0 // 4, h0, 0)
    b0, kk = _pick_bucket(hist, lane, kk)

    _zero_hist(hist)

    def h1(i, c):
        for u in range(4):
            s = load0(i * 4 + u)
            d0 = jnp.bitwise_xor((s >> 28) & 15, 8)
            d = (s >> 24) & 15
            plsc.addupdate_scatter(hist, [lane * 16 + d], ones, mask=d0 == b0)
        return c

    lax.fori_loop(0, n0 // 4, h1, 0)
    b1, kk = _pick_bucket(hist, lane, kk)

    _zero_hist(hist)

    def h2(i, c):
        for u in range(4):
            s = load0(i * 4 + u)
            d0 = jnp.bitwise_xor((s >> 28) & 15, 8)
            d1 = (s >> 24) & 15
            m = jnp.logical_and(d0 == b0, d1 == b1)
            m = jnp.logical_and(m, c < 1024)
            d = (s >> 20) & 15
            plsc.store_scatter(sa, [lane * 1024 + c], s, mask=m)
            plsc.addupdate_scatter(hist, [lane * 16 + d], ones, mask=m)
            c = c + m.astype(jnp.int32)
        return c

    cnt = lax.fori_loop(0, n0 // 4, h2, jnp.zeros((16,), jnp.int32))
    bkt, kk = _pick_bucket(hist, lane, kk)
    prefix = (jnp.bitwise_xor(b0, 8) << 28) | (b1 << 24) | (bkt << 20)

    src, dst = sa, sb
    for p in range(3, 8):
        shift = 28 - 4 * p
        shift_prev = shift + 4
        _zero_hist(hist)

        def fhp(i, c, src=src, dst=dst, bucket=bkt, shift=shift,
                shift_prev=shift_prev, cnt_in=cnt):
            s = plsc.load_gather(src, [lane * 1024 + i])
            valid = i < cnt_in
            m = jnp.logical_and(((s >> shift_prev) & 15) == bucket, valid)
            dnext = (s >> shift) & 15
            plsc.store_scatter(dst, [lane * 1024 + c], s, mask=m)
            plsc.addupdate_scatter(hist, [lane * 16 + dnext], ones, mask=m)
            return c + m.astype(jnp.int32)

        trip = jnp.max(cnt)
        cnt = lax.fori_loop(0, trip, fhp, jnp.zeros((16,), jnp.int32))
        src, dst = dst, src
        bkt, kk = _pick_bucket(hist, lane, kk)
        prefix = jnp.bitwise_or(prefix, bkt << shift)
    return prefix


def _sc_threshold_body(m1_hbm, pre2_hbm, t_hbm,
                       m1g, rowsv, sa, sb, hist, idxq, tout, dsem):
    cid = lax.axis_index("c")
    sid = lax.axis_index("s")
    wid = sid * 2 + cid
    base = wid * 256
    lane = lax.iota(jnp.int32, 16)
    zeros16 = jnp.zeros((16,), jnp.int32)

    def group(g, carry):
        row0 = base + g * 16
        pltpu.sync_copy(m1_hbm.at[pl.ds(row0, 16)], m1g)

        def load_a(i):
            return _f32_to_key(plsc.load_gather(m1g, [lane, zeros16 + i]))

        t30 = _radix_select(load_a, NCH, K_TOP, sa, sb, hist, lane)

        def qb(j, c):
            s = load_a(j)
            m = jnp.logical_and(s >= t30, c < K_TOP)
            pos = lane * K_TOP + c
            chunk = (row0 + lane) * NCH + j
            plsc.store_scatter(idxq, [pos // 120, pos % 120], chunk, mask=m)
            return c + m.astype(jnp.int32)

        lax.fori_loop(0, NCH, qb, zeros16)

        cps = [pltpu.async_copy(pre2_hbm.at[idxq.at[i]],
                                rowsv.at[pl.ds(i * 120, 120)], dsem)
               for i in range(4)]
        for cp in cps:
            cp.wait()

        def load_c(i):
            return _f32_to_key(
                plsc.load_gather(rowsv, [lane * K_TOP + (i >> 7),
                                         zeros16 + (i & 127)]))

        v30 = _radix_select(load_c, K_TOP * CHUNK, K_TOP, sa, sb, hist, lane)
        tout[pl.ds(g * 16, 16)] = _key_to_f32(v30)
        return carry

    lax.fori_loop(0, 16, group, 0)
    pltpu.sync_copy(tout, t_hbm.at[pl.ds(base, 256)])


@jax.jit
def kernel(x, W_enc, b_enc, W_dec, b_dec):
    B, D = x.shape
    F = W_enc.shape[0]
    BR = 512
    BF = 2048

    benc2 = b_enc.reshape(1, F)
    bdec2 = b_dec.reshape(1, D)
    W_decT = W_dec.T

    pre = pl.pallas_call(
        _encode_body,
        grid=(B // BR, F // BF),
        in_specs=[
            pl.BlockSpec((BR, D), lambda i, j: (i, 0)),
            pl.BlockSpec((BF, D), lambda i, j: (j, 0)),
            pl.BlockSpec((1, BF), lambda i, j: (0, j)),
            pl.BlockSpec((1, D), lambda i, j: (0, 0)),
        ],
        out_specs=pl.BlockSpec((BR, BF), lambda i, j: (i, j)),
        out_shape=jax.ShapeDtypeStruct((B, F), jnp.float32),
        compiler_params=pltpu.CompilerParams(
            dimension_semantics=("parallel", "parallel"),
        ),
    )(x, W_enc, benc2, bdec2)

    pre2 = pre.reshape(B * NCH, CHUNK)
    BR2 = 128
    m1 = pl.pallas_call(
        _chunkmax_body,
        grid=(B // BR2,),
        in_specs=[pl.BlockSpec((BR2, F), lambda i: (i, 0))],
        out_specs=pl.BlockSpec((BR2, NCH), lambda i: (i, 0)),
        out_shape=jax.ShapeDtypeStruct((B, NCH), jnp.float32),
        compiler_params=pltpu.CompilerParams(
            dimension_semantics=("arbitrary",),
        ),
    )(pre)
    mesh = plsc.VectorSubcoreMesh(core_axis_name="c", subcore_axis_name="s")
    t = pl.kernel(
        _sc_threshold_body,
        out_type=jax.ShapeDtypeStruct((B,), jnp.float32),
        mesh=mesh,
        compiler_params=pltpu.CompilerParams(use_tc_tiling_on_sc=False,
                                             needs_layout_passes=False),
        scratch_types=[
            pltpu.VMEM((16, NCH), jnp.float32),
            pltpu.VMEM((16 * K_TOP, CHUNK), jnp.float32),
            pltpu.VMEM((16 * 1024,), jnp.int32),
            pltpu.VMEM((16 * 1024,), jnp.int32),
            pltpu.VMEM((256,), jnp.int32),
            pltpu.VMEM((4, 120), jnp.int32),
            pltpu.VMEM((256,), jnp.float32),
            pltpu.SemaphoreType.DMA,
        ],
    )(m1, pre2)

    sparse, recon = pl.pallas_call(
        _decode_body,
        grid=(B // BR, F // BF),
        in_specs=[
            pl.BlockSpec((BR, BF), lambda i, j: (i, j)),
            pl.BlockSpec((BR, 1), lambda i, j: (i, 0)),
            pl.BlockSpec((BF, D), lambda i, j: (j, 0)),
            pl.BlockSpec((1, D), lambda i, j: (0, 0)),
        ],
        out_specs=[
            pl.BlockSpec((BR, BF), lambda i, j: (i, j)),
            pl.BlockSpec((BR, D), lambda i, j: (i, 0)),
        ],
        out_shape=[
            jax.ShapeDtypeStruct((B, F), jnp.float32),
            jax.ShapeDtypeStruct((B, D), jnp.float32),
        ],
        compiler_params=pltpu.CompilerParams(
            dimension_semantics=("parallel", "arbitrary"),
        ),
    )(pre, t.reshape(B, 1), W_decT, bdec2)

    return (recon, sparse)
